# baseline (speedup 1.0000x reference)
import jax
import jax.numpy as jnp
from jax import lax
from jax.experimental import pallas as pl
from jax.experimental.pallas import tpu as pltpu
from jax.experimental.pallas import tpu_sc as plsc

N = 128
V = 10000
E = 320000
LANES = 16
ROWS_PER_TILE = 4
CHUNK = 16000
NCHUNKS = E // CHUNK
CHUNKW = CHUNK // 2


def _sc_body(flat_hbm, cp_hbm, rp_hbm, wp_hbm, out_hbm,
             acc_v, xp_v, cp_b0, rp_b0, wp_b0, cp_b1, rp_b1, wp_b1,
             sem0, sem1, xsem):
    nc = plsc.get_sparse_core_info().num_cores
    wid = lax.axis_index("s") * nc + lax.axis_index("c")
    base = wid * ROWS_PER_TILE * V

    bufs = ((cp_b0, rp_b0, wp_b0, sem0), (cp_b1, rp_b1, wp_b1, sem1))

    def start(slot, w0):
        cp_v, rp_v, wp_v, sem = bufs[slot]
        pltpu.async_copy(cp_hbm.at[pl.ds(w0, CHUNKW)], cp_v, sem)
        pltpu.async_copy(rp_hbm.at[pl.ds(w0, CHUNKW)], rp_v, sem)
        pltpu.async_copy(wp_hbm.at[pl.ds(w0, CHUNKW)], wp_v, sem)

    def wait(slot):
        cp_v, rp_v, wp_v, sem = bufs[slot]
        pltpu.make_async_copy(cp_hbm.at[pl.ds(0, CHUNKW)], cp_v, sem).wait()
        pltpu.make_async_copy(rp_hbm.at[pl.ds(0, CHUNKW)], rp_v, sem).wait()
        pltpu.make_async_copy(wp_hbm.at[pl.ds(0, CHUNKW)], wp_v, sem).wait()

    start(0, 0)
    start(1, CHUNKW)
    pltpu.async_copy(flat_hbm.at[pl.ds(base, ROWS_PER_TILE * V)], acc_v, xsem)
    pltpu.make_async_copy(flat_hbm.at[pl.ds(0, ROWS_PER_TILE * V)], acc_v, xsem).wait()

    @plsc.parallel_loop(0, V, LANES, unroll=8)
    def _pk(i):
        for j2 in range(ROWS_PER_TILE // 2):
            a = acc_v[pl.ds(2 * j2 * V + i, LANES)]
            b = acc_v[pl.ds((2 * j2 + 1) * V + i, LANES)]
            ab = plsc.pack(a, b, format=plsc.PackFormat.INTERLEAVED)
            xp_v[pl.ds(j2 * V + i, LANES)] = plsc.bitcast(ab, jnp.int32)

    xp_rows = [xp_v.at[pl.ds(j2 * V, V)] for j2 in range(ROWS_PER_TILE // 2)]
    acc_rows = [acc_v.at[pl.ds(j * V, V)] for j in range(ROWS_PER_TILE)]

    def process(slot):
        cp_v, rp_v, wp_v, _ = bufs[slot]

        @plsc.parallel_loop(0, CHUNKW, LANES, unroll=4)
        def _grp(i):
            cp = cp_v[pl.ds(i, LANES)]
            rp = rp_v[pl.ds(i, LANES)]
            wp = wp_v[pl.ds(i, LANES)]
            cc = plsc.unpack(plsc.bitcast(cp, jnp.int16),
                             format=plsc.PackFormat.INTERLEAVED,
                             preferred_element_type=jnp.int32)
            rr = plsc.unpack(plsc.bitcast(rp, jnp.int16),
                             format=plsc.PackFormat.INTERLEAVED,
                             preferred_element_type=jnp.int32)
            ww = plsc.unpack(plsc.bitcast(wp, jnp.bfloat16),
                             format=plsc.PackFormat.INTERLEAVED)
            for h in range(2):
                c, r, w = cc[h], rr[h], ww[h]
                for j2 in range(ROWS_PER_TILE // 2):
                    gw = plsc.load_gather(xp_rows[j2], [c])
                    a, b = plsc.unpack(plsc.bitcast(gw, jnp.bfloat16),
                                       format=plsc.PackFormat.INTERLEAVED)
                    plsc.addupdate_scatter(acc_rows[2 * j2], [r], a * w)
                    plsc.addupdate_scatter(acc_rows[2 * j2 + 1], [r], b * w)

    @pl.loop(0, NCHUNKS, step=2)
    def _pair(g):
        wait(0)
        process(0)

        @pl.when(g + 2 < NCHUNKS)
        def _():
            start(0, (g + 2) * CHUNKW)

        wait(1)
        process(1)

        @pl.when(g + 3 < NCHUNKS)
        def _():
            start(1, (g + 3) * CHUNKW)

    pltpu.sync_copy(acc_v, out_hbm.at[pl.ds(base, ROWS_PER_TILE * V)])


@jax.jit
def _structured_logits_sc(flat, cpair, rpair, wpair):
    flat = flat.reshape(-1)
    mesh = plsc.VectorSubcoreMesh(core_axis_name="c", subcore_axis_name="s")
    return pl.kernel(
        _sc_body,
        out_type=jax.ShapeDtypeStruct((N * V,), jnp.float32),
        mesh=mesh,
        compiler_params=pltpu.CompilerParams(needs_layout_passes=False),
        scratch_types=[
            pltpu.VMEM((ROWS_PER_TILE * V,), jnp.float32),
            pltpu.VMEM((ROWS_PER_TILE // 2 * V,), jnp.int32),
            pltpu.VMEM((CHUNKW,), jnp.int32),
            pltpu.VMEM((CHUNKW,), jnp.int32),
            pltpu.VMEM((CHUNKW,), jnp.int32),
            pltpu.VMEM((CHUNKW,), jnp.int32),
            pltpu.VMEM((CHUNKW,), jnp.int32),
            pltpu.VMEM((CHUNKW,), jnp.int32),
            pltpu.SemaphoreType.DMA,
            pltpu.SemaphoreType.DMA,
            pltpu.SemaphoreType.DMA,
        ],
    )(flat, cpair, rpair, wpair)


def kernel(logits, edge_index, edge_values):
    old_shape = logits.shape
    flat = logits.reshape(-1, old_shape[-1])
    col2 = edge_index[1].astype(jnp.int16).reshape(-1, 2)
    row2 = edge_index[0].astype(jnp.int16).reshape(-1, 2)
    vv2 = edge_values.astype(jnp.bfloat16).reshape(-1, 2)
    cpair = lax.bitcast_convert_type(col2, jnp.int32)
    rpair = lax.bitcast_convert_type(row2, jnp.int32)
    wpair = lax.bitcast_convert_type(vv2, jnp.int32)
    out = _structured_logits_sc(flat, cpair, rpair, wpair)
    return out.reshape(old_shape)
